# baseline (speedup 1.0000x reference)
import jax
import jax.numpy as jnp
from jax.experimental import pallas as pl

B = 16384
D = 128
GRID = 20
BLK = 2048


def _fused(x_ref, we0, be0, we1, be1, we2, be2, we3, be3,
           wd0, bd0, wd1, bd1, wd2, bd2, wd3, bd3,
           ew0, eb0, ew1, eb1, somw,
           code_out, xp_out, cosim_out, z_out, gamma_out):
    eps = 1e-8
    x = x_ref[...]
    h = jnp.tanh(x @ we0[...] + be0[...])
    h = jnp.tanh(h @ we1[...] + be1[...])
    h = jnp.tanh(h @ we2[...] + be2[...])
    code = h @ we3[...] + be3[...]
    g = jnp.tanh(code @ wd0[...] + bd0[...])
    g = jnp.tanh(g @ wd1[...] + bd1[...])
    g = jnp.tanh(g @ wd2[...] + bd2[...])
    xp = g @ wd3[...] + bd3[...]

    dot = jnp.sum(x * xp, axis=1)
    nx2 = jnp.sum(x * x, axis=1)
    nx = jnp.sqrt(nx2)
    nxp = jnp.sqrt(jnp.sum(xp * xp, axis=1))
    cosim = dot / (nx * nxp + eps)
    euclid = jnp.sqrt(jnp.sum((x - xp) ** 2, axis=1)) / (nx + eps)

    sw = somw[...]
    d2 = (nx2[:, None]
          - 2.0 * (x @ sw.T)
          + jnp.sum(sw * sw, axis=1)[None, :])
    idx = jnp.argmin(d2, axis=1)
    zi = (idx // GRID).astype(jnp.float32)
    zj = (idx % GRID).astype(jnp.float32)

    z = jnp.concatenate([code, cosim[:, None], euclid[:, None],
                         zi[:, None] / 20.0, zj[:, None] / 20.0], axis=1)

    e = jnp.tanh(z @ ew0[...] + eb0[...])
    logits = e @ ew1[...] + eb1[...]
    gamma = jax.nn.softmax(logits, axis=1)

    code_out[...] = code
    xp_out[...] = xp
    cosim_out[...] = cosim
    z_out[...] = z
    gamma_out[...] = gamma


def kernel(X, We0, be0, We1, be1, We2, be2, We3, be3,
           Wd0, bd0, Wd1, bd1, Wd2, bd2, Wd3, bd3,
           Ew0, Eb0, Ew1, Eb1, som_w):
    f32 = jnp.float32
    grid = B // BLK

    def full(a):
        return pl.BlockSpec(a.shape, lambda i: (0,) * a.ndim)

    biases = [b.reshape(1, -1) for b in (be0, be1, be2, be3,
                                         bd0, bd1, bd2, bd3, Eb0, Eb1)]
    (be0r, be1r, be2r, be3r, bd0r, bd1r, bd2r, bd3r, eb0r, eb1r) = biases

    in_arrays = (X, We0, be0r, We1, be1r, We2, be2r, We3, be3r,
                 Wd0, bd0r, Wd1, bd1r, Wd2, bd2r, Wd3, bd3r,
                 Ew0, eb0r, Ew1, eb1r, som_w)
    in_specs = [pl.BlockSpec((BLK, D), lambda i: (i, 0))]
    in_specs += [full(a) for a in in_arrays[1:]]

    out_shape = (
        jax.ShapeDtypeStruct((B, 2), f32),
        jax.ShapeDtypeStruct((B, D), f32),
        jax.ShapeDtypeStruct((B,), f32),
        jax.ShapeDtypeStruct((B, 6), f32),
        jax.ShapeDtypeStruct((B, 4), f32),
    )
    out_specs = (
        pl.BlockSpec((BLK, 2), lambda i: (i, 0)),
        pl.BlockSpec((BLK, D), lambda i: (i, 0)),
        pl.BlockSpec((BLK,), lambda i: (i,)),
        pl.BlockSpec((BLK, 6), lambda i: (i, 0)),
        pl.BlockSpec((BLK, 4), lambda i: (i, 0)),
    )

    return pl.pallas_call(
        _fused,
        grid=(grid,),
        in_specs=in_specs,
        out_specs=out_specs,
        out_shape=out_shape,
    )(*in_arrays)


# speedup vs baseline: 1.2841x; 1.2841x over previous
import jax
import jax.numpy as jnp
from jax.experimental import pallas as pl

B = 16384
D = 128
GRID = 20
BLK = 2048


def _fused(x_ref, we0, be0, we1, be1, we2, be2, we3, be3,
           wd0, bd0, wd1, bd1, wd2, bd2, wd3, bd3,
           ew0, eb0, ew1, eb1, somw, sel3,
           code_out, xp_out, cosim_out, z_out, gamma_out):
    eps = 1e-8
    xT = x_ref[...].T
    h = jnp.tanh(we0[...] @ xT + be0[...])
    h = jnp.tanh(we1[...] @ h + be1[...])
    h = jnp.tanh(we2[...] @ h + be2[...])
    codeT = we3[...] @ h + be3[...]
    g = jnp.tanh(wd0[...] @ codeT + bd0[...])
    g = jnp.tanh(wd1[...] @ g + bd1[...])
    g = jnp.tanh(wd2[...] @ g + bd2[...])
    xpT = wd3[...] @ g + bd3[...]

    prods = jnp.concatenate([xT * xT, xT * xpT, xpT * xpT], axis=0)
    sums = sel3[...] @ prods
    nx2 = sums[0:1, :]
    dot = sums[1:2, :]
    nxp2 = sums[2:3, :]
    nx = jnp.sqrt(nx2)
    cosim = dot / (nx * jnp.sqrt(nxp2) + eps)
    euclid = jnp.sqrt(jnp.maximum(nx2 - 2.0 * dot + nxp2, 0.0)) / (nx + eps)

    sw = somw[...]
    swsq = (sw * sw) @ jnp.ones((D, 1), jnp.float32)
    d2 = swsq - 2.0 * (sw @ xT)
    idx = jnp.argmin(d2, axis=0).reshape(1, BLK)
    zi = (idx // GRID).astype(jnp.float32)
    zj = (idx % GRID).astype(jnp.float32)

    zT = jnp.concatenate([codeT, cosim, euclid,
                          zi / 20.0, zj / 20.0], axis=0)

    e = jnp.tanh(ew0[...] @ zT + eb0[...])
    logits = ew1[...] @ e + eb1[...]
    m = jnp.max(logits, axis=0, keepdims=True)
    ex = jnp.exp(logits - m)
    gammaT = ex / jnp.sum(ex, axis=0, keepdims=True)

    tail = jnp.concatenate([zT, gammaT, cosim], axis=0).T
    xp_out[...] = xpT.T
    code_out[...] = tail[:, 0:2]
    z_out[...] = tail[:, 0:6]
    gamma_out[...] = tail[:, 6:10]
    cosim_out[...] = tail[:, 10]


def kernel(X, We0, be0, We1, be1, We2, be2, We3, be3,
           Wd0, bd0, Wd1, bd1, Wd2, bd2, Wd3, bd3,
           Ew0, Eb0, Ew1, Eb1, som_w):
    f32 = jnp.float32
    grid = B // BLK

    wTs = [w.T for w in (We0, We1, We2, We3, Wd0, Wd1, Wd2, Wd3, Ew0, Ew1)]
    bTs = [b.reshape(-1, 1) for b in (be0, be1, be2, be3,
                                      bd0, bd1, bd2, bd3, Eb0, Eb1)]
    (we0T, we1T, we2T, we3T, wd0T, wd1T, wd2T, wd3T, ew0T, ew1T) = wTs
    (be0c, be1c, be2c, be3c, bd0c, bd1c, bd2c, bd3c, eb0c, eb1c) = bTs
    sel3 = jnp.repeat(jnp.eye(3, dtype=f32), D, axis=1)

    def full(a):
        return pl.BlockSpec(a.shape, lambda i: (0,) * a.ndim)

    in_arrays = (X, we0T, be0c, we1T, be1c, we2T, be2c, we3T, be3c,
                 wd0T, bd0c, wd1T, bd1c, wd2T, bd2c, wd3T, bd3c,
                 ew0T, eb0c, ew1T, eb1c, som_w, sel3)
    in_specs = [pl.BlockSpec((BLK, D), lambda i: (i, 0))]
    in_specs += [full(a) for a in in_arrays[1:]]

    out_shape = (
        jax.ShapeDtypeStruct((B, 2), f32),
        jax.ShapeDtypeStruct((B, D), f32),
        jax.ShapeDtypeStruct((B,), f32),
        jax.ShapeDtypeStruct((B, 6), f32),
        jax.ShapeDtypeStruct((B, 4), f32),
    )
    out_specs = (
        pl.BlockSpec((BLK, 2), lambda i: (i, 0)),
        pl.BlockSpec((BLK, D), lambda i: (i, 0)),
        pl.BlockSpec((BLK,), lambda i: (i,)),
        pl.BlockSpec((BLK, 6), lambda i: (i, 0)),
        pl.BlockSpec((BLK, 4), lambda i: (i, 0)),
    )

    return pl.pallas_call(
        _fused,
        grid=(grid,),
        in_specs=in_specs,
        out_specs=out_specs,
        out_shape=out_shape,
    )(*in_arrays)
